# baseline (speedup 1.0000x reference)
import jax
import jax.numpy as jnp
from jax import lax
from jax.experimental import pallas as pl
from jax.experimental.pallas import tpu as pltpu
from jax.experimental.pallas import tpu_sc as plsc

N = 10000
E = 320000
D = 128

NC = 2
NS = 16
NW = NC * NS
EPW = E // NW
CHUNK = 125
KCH = EPW // CHUNK
NPAD = 10240
RPT = NPAD // NS


def _sc_agg_body(feat_hbm, src_hbm, dst_hbm, zeros_hbm, out_hbm,
                 s0, s1, s2, s3, didx, rows0, rows1,
                 is0, is1, is2, is3, gs0, gs1, acc):
    c = lax.axis_index("c")
    s = lax.axis_index("s")
    wid = s * NC + c

    pltpu.sync_copy(dst_hbm.at[1, wid], didx)
    pltpu.sync_copy(zeros_hbm, acc.at[pl.ds(s * RPT, RPT)])
    plsc.subcore_barrier()

    pltpu.sync_copy(src_hbm.at[0, wid, 0], s0)
    pltpu.sync_copy(src_hbm.at[0, wid, 1], s1)
    pltpu.async_copy(feat_hbm.at[s0.at[0]], rows0, gs0)
    pltpu.async_copy(feat_hbm.at[s1.at[0]], rows1, gs1)
    pltpu.async_copy(src_hbm.at[0, wid, 2], s2, is2)
    pltpu.async_copy(src_hbm.at[0, wid, 3], s3, is3)

    def body(t, _):
        q0 = 4 * t
        nl = t + 1 < KCH // 4
        pltpu.make_async_copy(feat_hbm.at[s0.at[0]], rows0, gs0).wait()
        pltpu.sync_copy(rows0, acc.at[didx.at[q0]], add=True)
        pltpu.make_async_copy(src_hbm.at[0, wid, q0 + 2], s2, is2).wait()
        pltpu.async_copy(feat_hbm.at[s2.at[0]], rows0, gs0)

        @pl.when(nl)
        def _():
            pltpu.async_copy(src_hbm.at[0, wid, q0 + 4], s0, is0)

        pltpu.make_async_copy(feat_hbm.at[s1.at[0]], rows1, gs1).wait()
        pltpu.sync_copy(rows1, acc.at[didx.at[q0 + 1]], add=True)
        pltpu.make_async_copy(src_hbm.at[0, wid, q0 + 3], s3, is3).wait()
        pltpu.async_copy(feat_hbm.at[s3.at[0]], rows1, gs1)

        @pl.when(nl)
        def _():
            pltpu.async_copy(src_hbm.at[0, wid, q0 + 5], s1, is1)

        pltpu.make_async_copy(feat_hbm.at[s2.at[0]], rows0, gs0).wait()
        pltpu.sync_copy(rows0, acc.at[didx.at[q0 + 2]], add=True)

        @pl.when(nl)
        def _():
            pltpu.make_async_copy(src_hbm.at[0, wid, q0 + 4], s0, is0).wait()
            pltpu.async_copy(feat_hbm.at[s0.at[0]], rows0, gs0)
            pltpu.async_copy(src_hbm.at[0, wid, q0 + 6], s2, is2)

        pltpu.make_async_copy(feat_hbm.at[s3.at[0]], rows1, gs1).wait()
        pltpu.sync_copy(rows1, acc.at[didx.at[q0 + 3]], add=True)

        @pl.when(nl)
        def _():
            pltpu.make_async_copy(src_hbm.at[0, wid, q0 + 5], s1, is1).wait()
            pltpu.async_copy(feat_hbm.at[s1.at[0]], rows1, gs1)
            pltpu.async_copy(src_hbm.at[0, wid, q0 + 7], s3, is3)

        return 0

    lax.fori_loop(0, KCH // 4, body, 0)
    plsc.subcore_barrier()

    pltpu.sync_copy(acc.at[pl.ds(s * RPT, RPT)],
                    out_hbm.at[c, pl.ds(s * RPT, RPT)])


@jax.jit
def _sc_agg(feat, src4, dst3, zeros):
    mesh = plsc.VectorSubcoreMesh(core_axis_name="c", subcore_axis_name="s",
                                  num_cores=NC, num_subcores=NS)
    f = pl.kernel(
        _sc_agg_body,
        out_type=jax.ShapeDtypeStruct((NC, NPAD, D), jnp.float32),
        mesh=mesh,
        scratch_types=[
            pltpu.VMEM((1, CHUNK), jnp.int32),
            pltpu.VMEM((1, CHUNK), jnp.int32),
            pltpu.VMEM((1, CHUNK), jnp.int32),
            pltpu.VMEM((1, CHUNK), jnp.int32),
            pltpu.VMEM((KCH, CHUNK), jnp.int32),
            pltpu.VMEM((CHUNK, D), jnp.float32),
            pltpu.VMEM((CHUNK, D), jnp.float32),
            pltpu.SemaphoreType.DMA,
            pltpu.SemaphoreType.DMA,
            pltpu.SemaphoreType.DMA,
            pltpu.SemaphoreType.DMA,
            pltpu.SemaphoreType.DMA,
            pltpu.SemaphoreType.DMA,
            pltpu.VMEM_SHARED((NPAD, D), jnp.float32),
        ],
    )
    return f(feat, src4, dst3, zeros)


def _tc_self_body(feat_ref, ws_ref, out_ref):
    out_ref[...] = lax.dot(feat_ref[...], ws_ref[...],
                           preferred_element_type=jnp.float32)


@jax.jit
def _tc_self(feat, ws):
    blk = 2000
    return pl.pallas_call(
        _tc_self_body,
        grid=(N // blk,),
        in_specs=[
            pl.BlockSpec((blk, D), lambda i: (i, 0)),
            pl.BlockSpec((D, D), lambda i: (0, 0)),
        ],
        out_specs=pl.BlockSpec((blk, D), lambda i: (i, 0)),
        out_shape=jax.ShapeDtypeStruct((N, D), jnp.float32),
    )(feat, ws)


def _tc_combine_body(p_ref, base_ref, norm_ref, wn_ref, out_ref):
    ssum = p_ref[0] + p_ref[1]
    agg = lax.dot(ssum, wn_ref[...], preferred_element_type=jnp.float32)
    out_ref[...] = agg * norm_ref[...] + base_ref[...]


@jax.jit
def _tc_combine(partials, base, norm, wn):
    blk = 1000
    return pl.pallas_call(
        _tc_combine_body,
        grid=(N // blk,),
        in_specs=[
            pl.BlockSpec((NC, blk, D), lambda i: (0, i, 0)),
            pl.BlockSpec((blk, D), lambda i: (i, 0)),
            pl.BlockSpec((blk, 1), lambda i: (i, 0)),
            pl.BlockSpec((D, D), lambda i: (0, 0)),
        ],
        out_specs=pl.BlockSpec((blk, D), lambda i: (i, 0)),
        out_shape=jax.ShapeDtypeStruct((N, D), jnp.float32),
    )(partials, base, norm, wn)


def kernel(feat, edge_index, norm_data, weight_n, weight_s):
    ei = edge_index.astype(jnp.int32)
    src5 = ei.reshape(2, NW, KCH, 1, CHUNK)
    dst4 = ei.reshape(2, NW, KCH, CHUNK)
    zeros = jnp.zeros((RPT, D), jnp.float32)
    base = _tc_self(feat, weight_s)
    partials = _sc_agg(feat, src5, dst4, zeros)
    return _tc_combine(partials, base, norm_data, weight_n)


# speedup vs baseline: 1.0856x; 1.0856x over previous
import jax
import jax.numpy as jnp
from jax import lax
from jax.experimental import pallas as pl
from jax.experimental.pallas import tpu as pltpu
from jax.experimental.pallas import tpu_sc as plsc

N = 10000
E = 320000
D = 128

NC = 2
NS = 16
NW = NC * NS
EPW = E // NW
CHUNK = 125
KCH = EPW // CHUNK
NPAD = 10240
RPT = NPAD // NS


def _sc_agg_body(feat_hbm, edge_hbm, zeros_hbm, out_hbm,
                 sidx, didx, rows0, rows1, isem, gs0, gs1, acc):
    c = lax.axis_index("c")
    s = lax.axis_index("s")
    wid = s * NC + c

    pltpu.sync_copy(edge_hbm.at[1, wid], didx)
    pltpu.sync_copy(zeros_hbm, acc.at[pl.ds(s * RPT, RPT)])

    pltpu.sync_copy(edge_hbm.at[0, wid, pl.ds(0, 16)], sidx)
    plsc.subcore_barrier()

    pltpu.async_copy(feat_hbm.at[sidx.at[0]], rows0, gs0)
    pltpu.async_copy(feat_hbm.at[sidx.at[1]], rows1, gs1)

    def _refresh_slices(t):
        gl = t // 2 + 1
        goff = pl.multiple_of(8 * gl, 8)
        roff = pl.multiple_of((gl % 2) * 8, 8)
        return (edge_hbm.at[0, wid, pl.ds(goff, 8)],
                sidx.at[pl.ds(roff, 8)])

    def body(t, _):
        q0 = 4 * t
        rb = (t % 4) * 4

        @pl.when(jnp.logical_and(t % 2 == 0,
                                 jnp.logical_and(t >= 2, t <= 16)))
        def _():
            src_slc, dst_slc = _refresh_slices(t)
            pltpu.async_copy(src_slc, dst_slc, isem)

        def section(k, rbuf, gsem):
            q = q0 + k
            pltpu.make_async_copy(feat_hbm.at[sidx.at[rb + k]],
                                  rbuf, gsem).wait()
            pltpu.sync_copy(rbuf, acc.at[didx.at[q]], add=True)

            if k == 2:
                @pl.when(jnp.logical_and(t % 2 == 1,
                                         jnp.logical_and(t >= 3, t <= 17)))
                def _():
                    src_slc, dst_slc = _refresh_slices(t - 1)
                    pltpu.make_async_copy(src_slc, dst_slc, isem).wait()

            @pl.when(q + 2 < KCH)
            def _():
                rk2 = lax.rem(rb + k + 2, 16)
                pltpu.async_copy(feat_hbm.at[sidx.at[rk2]], rbuf, gsem)

        section(0, rows0, gs0)
        section(1, rows1, gs1)
        section(2, rows0, gs0)
        section(3, rows1, gs1)
        return 0

    lax.fori_loop(0, KCH // 4, body, 0)
    plsc.subcore_barrier()

    pltpu.sync_copy(acc.at[pl.ds(s * RPT, RPT)],
                    out_hbm.at[c, pl.ds(s * RPT, RPT)])


@jax.jit
def _sc_agg(feat, edge4, zeros):
    mesh = plsc.VectorSubcoreMesh(core_axis_name="c", subcore_axis_name="s",
                                  num_cores=NC, num_subcores=NS)
    f = pl.kernel(
        _sc_agg_body,
        out_type=jax.ShapeDtypeStruct((NC, NPAD, D), jnp.float32),
        mesh=mesh,
        scratch_types=[
            pltpu.VMEM((16, CHUNK), jnp.int32),
            pltpu.VMEM((KCH, CHUNK), jnp.int32),
            pltpu.VMEM((CHUNK, D), jnp.float32),
            pltpu.VMEM((CHUNK, D), jnp.float32),
            pltpu.SemaphoreType.DMA,
            pltpu.SemaphoreType.DMA,
            pltpu.SemaphoreType.DMA,
            pltpu.VMEM_SHARED((NPAD, D), jnp.float32),
        ],
    )
    return f(feat, edge4, zeros)


def _tc_self_body(feat_ref, ws_ref, out_ref):
    out_ref[...] = lax.dot(feat_ref[...], ws_ref[...],
                           preferred_element_type=jnp.float32)


@jax.jit
def _tc_self(feat, ws):
    blk = 2000
    return pl.pallas_call(
        _tc_self_body,
        grid=(N // blk,),
        in_specs=[
            pl.BlockSpec((blk, D), lambda i: (i, 0)),
            pl.BlockSpec((D, D), lambda i: (0, 0)),
        ],
        out_specs=pl.BlockSpec((blk, D), lambda i: (i, 0)),
        out_shape=jax.ShapeDtypeStruct((N, D), jnp.float32),
    )(feat, ws)


def _tc_combine_body(p_ref, base_ref, norm_ref, wn_ref, out_ref):
    ssum = p_ref[0] + p_ref[1]
    agg = lax.dot(ssum, wn_ref[...], preferred_element_type=jnp.float32)
    out_ref[...] = agg * norm_ref[...] + base_ref[...]


@jax.jit
def _tc_combine(partials, base, norm, wn):
    blk = 1000
    return pl.pallas_call(
        _tc_combine_body,
        grid=(N // blk,),
        in_specs=[
            pl.BlockSpec((NC, blk, D), lambda i: (0, i, 0)),
            pl.BlockSpec((blk, D), lambda i: (i, 0)),
            pl.BlockSpec((blk, 1), lambda i: (i, 0)),
            pl.BlockSpec((D, D), lambda i: (0, 0)),
        ],
        out_specs=pl.BlockSpec((blk, D), lambda i: (i, 0)),
        out_shape=jax.ShapeDtypeStruct((N, D), jnp.float32),
    )(partials, base, norm, wn)


def kernel(feat, edge_index, norm_data, weight_n, weight_s):
    edge4 = edge_index.astype(jnp.int32).reshape(2, NW, KCH, CHUNK)
    zeros = jnp.zeros((RPT, D), jnp.float32)
    base = _tc_self(feat, weight_s)
    partials = _sc_agg(feat, edge4, zeros)
    return _tc_combine(partials, base, norm_data, weight_n)


# speedup vs baseline: 1.0944x; 1.0081x over previous
import jax
import jax.numpy as jnp
from jax import lax
from jax.experimental import pallas as pl
from jax.experimental.pallas import tpu as pltpu
from jax.experimental.pallas import tpu_sc as plsc

N = 10000
E = 320000
D = 128

NC = 2
NS = 16
NW = NC * NS
EPW = E // NW
CHUNK = 125
KCH = EPW // CHUNK
NPAD = 10240
RPT = NPAD // NS


def _sc_agg_body(feat_hbm, edge_hbm, zeros_hbm, out_hbm,
                 sidx, didx, rows0, rows1, isem, gs0, gs1, acc):
    c = lax.axis_index("c")
    s = lax.axis_index("s")
    wid = s * NC + c

    pltpu.sync_copy(edge_hbm.at[1, wid], didx)
    pltpu.sync_copy(zeros_hbm, acc.at[pl.ds(s * RPT, RPT)])

    pltpu.sync_copy(edge_hbm.at[0, wid, pl.ds(0, 16)], sidx)
    plsc.subcore_barrier()

    pltpu.async_copy(feat_hbm.at[sidx.at[0]], rows0, gs0)
    pltpu.async_copy(feat_hbm.at[sidx.at[1]], rows1, gs1)

    def _refresh_slices(t):
        gl = t // 2 + 1
        goff = pl.multiple_of(8 * gl, 8)
        roff = pl.multiple_of((gl % 2) * 8, 8)
        return (edge_hbm.at[0, wid, pl.ds(goff, 8)],
                sidx.at[pl.ds(roff, 8)])

    def body(t, _):
        q0 = 4 * t
        rb = (t % 4) * 4

        @pl.when(jnp.logical_and(t % 2 == 0,
                                 jnp.logical_and(t >= 2, t <= 16)))
        def _():
            src_slc, dst_slc = _refresh_slices(t)
            pltpu.async_copy(src_slc, dst_slc, isem)

        def section(k, rbuf, gsem):
            q = q0 + k
            pltpu.make_async_copy(feat_hbm.at[sidx.at[rb + k]],
                                  rbuf, gsem).wait()
            pltpu.sync_copy(rbuf, acc.at[didx.at[q]], add=True)

            if k == 2:
                @pl.when(jnp.logical_and(t % 2 == 1,
                                         jnp.logical_and(t >= 3, t <= 17)))
                def _():
                    src_slc, dst_slc = _refresh_slices(t - 1)
                    pltpu.make_async_copy(src_slc, dst_slc, isem).wait()

            @pl.when(q + 2 < KCH)
            def _():
                rk2 = lax.rem(rb + k + 2, 16)
                pltpu.async_copy(feat_hbm.at[sidx.at[rk2]], rbuf, gsem)

        section(0, rows0, gs0)
        section(1, rows1, gs1)
        section(2, rows0, gs0)
        section(3, rows1, gs1)
        return 0

    lax.fori_loop(0, KCH // 4, body, 0)
    plsc.subcore_barrier()

    pltpu.sync_copy(acc.at[pl.ds(s * RPT, RPT)],
                    out_hbm.at[c, pl.ds(s * RPT, RPT)])


@jax.jit
def _sc_agg(feat, edge4, zeros):
    mesh = plsc.VectorSubcoreMesh(core_axis_name="c", subcore_axis_name="s",
                                  num_cores=NC, num_subcores=NS)
    f = pl.kernel(
        _sc_agg_body,
        out_type=jax.ShapeDtypeStruct((NC, NPAD, D), jnp.float32),
        mesh=mesh,
        scratch_types=[
            pltpu.VMEM((16, CHUNK), jnp.int32),
            pltpu.VMEM((KCH, CHUNK), jnp.int32),
            pltpu.VMEM((CHUNK, D), jnp.float32),
            pltpu.VMEM((CHUNK, D), jnp.float32),
            pltpu.SemaphoreType.DMA,
            pltpu.SemaphoreType.DMA,
            pltpu.SemaphoreType.DMA,
            pltpu.VMEM_SHARED((NPAD, D), jnp.float32),
        ],
    )
    return f(feat, edge4, zeros)


def _tc_self_body(feat_ref, ws_ref, out_ref):
    out_ref[...] = lax.dot(feat_ref[...], ws_ref[...],
                           preferred_element_type=jnp.float32)


@jax.jit
def _tc_self(feat, ws):
    blk = 2000
    return pl.pallas_call(
        _tc_self_body,
        grid=(N // blk,),
        in_specs=[
            pl.BlockSpec((blk, D), lambda i: (i, 0)),
            pl.BlockSpec((D, D), lambda i: (0, 0)),
        ],
        out_specs=pl.BlockSpec((blk, D), lambda i: (i, 0)),
        out_shape=jax.ShapeDtypeStruct((N, D), jnp.float32),
    )(feat, ws)


def _tc_combine_body(p_ref, base_ref, norm_ref, wn_ref, out_ref):
    ssum = p_ref[0] + p_ref[1]
    agg = lax.dot(ssum, wn_ref[...], preferred_element_type=jnp.float32)
    out_ref[...] = agg * norm_ref[...] + base_ref[...]


@jax.jit
def _tc_combine(partials, base, norm, wn):
    blk = 2000
    return pl.pallas_call(
        _tc_combine_body,
        grid=(N // blk,),
        in_specs=[
            pl.BlockSpec((NC, blk, D), lambda i: (0, i, 0)),
            pl.BlockSpec((blk, D), lambda i: (i, 0)),
            pl.BlockSpec((blk, 1), lambda i: (i, 0)),
            pl.BlockSpec((D, D), lambda i: (0, 0)),
        ],
        out_specs=pl.BlockSpec((blk, D), lambda i: (i, 0)),
        out_shape=jax.ShapeDtypeStruct((N, D), jnp.float32),
    )(partials, base, norm, wn)


def kernel(feat, edge_index, norm_data, weight_n, weight_s):
    edge4 = edge_index.astype(jnp.int32).reshape(2, NW, KCH, CHUNK)
    zeros = jnp.zeros((RPT, D), jnp.float32)
    base = _tc_self(feat, weight_s)
    partials = _sc_agg(feat, edge4, zeros)
    return _tc_combine(partials, base, norm_data, weight_n)


# speedup vs baseline: 1.1132x; 1.0172x over previous
import jax
import jax.numpy as jnp
from jax import lax
from jax.experimental import pallas as pl
from jax.experimental.pallas import tpu as pltpu
from jax.experimental.pallas import tpu_sc as plsc

N = 10000
E = 320000
D = 128

NC = 2
NS = 16
NW = NC * NS
EPW = E // NW
CHUNK = 125
KCH = EPW // CHUNK
NPAD = 10240
RPT = NPAD // NS


def _sc_agg_body(feat_hbm, edge_hbm, zeros_hbm, out_hbm,
                 sidx, didx, rows0, rows1, isem, gs0, gs1, acc):
    c = lax.axis_index("c")
    s = lax.axis_index("s")
    wid = s * NC + c

    pltpu.sync_copy(edge_hbm.at[0, wid, pl.ds(0, 16)], sidx)
    pltpu.async_copy(feat_hbm.at[sidx.at[0]], rows0, gs0)
    pltpu.async_copy(feat_hbm.at[sidx.at[1]], rows1, gs1)

    pltpu.sync_copy(edge_hbm.at[1, wid], didx)
    pltpu.sync_copy(zeros_hbm, acc.at[pl.ds(s * RPT, RPT)])
    plsc.subcore_barrier()

    def _refresh_slices(t):
        gl = t // 2 + 1
        goff = pl.multiple_of(8 * gl, 8)
        roff = pl.multiple_of((gl % 2) * 8, 8)
        return (edge_hbm.at[0, wid, pl.ds(goff, 8)],
                sidx.at[pl.ds(roff, 8)])

    def body(t, _):
        q0 = 4 * t
        rb = (t % 4) * 4

        @pl.when(jnp.logical_and(t % 2 == 0,
                                 jnp.logical_and(t >= 2, t <= 16)))
        def _():
            src_slc, dst_slc = _refresh_slices(t)
            pltpu.async_copy(src_slc, dst_slc, isem)

        def section(k, rbuf, gsem):
            q = q0 + k
            pltpu.make_async_copy(feat_hbm.at[sidx.at[rb + k]],
                                  rbuf, gsem).wait()
            pltpu.sync_copy(rbuf, acc.at[didx.at[q]], add=True)

            if k == 2:
                @pl.when(jnp.logical_and(t % 2 == 1,
                                         jnp.logical_and(t >= 3, t <= 17)))
                def _():
                    src_slc, dst_slc = _refresh_slices(t - 1)
                    pltpu.make_async_copy(src_slc, dst_slc, isem).wait()

            @pl.when(q + 2 < KCH)
            def _():
                rk2 = lax.rem(rb + k + 2, 16)
                pltpu.async_copy(feat_hbm.at[sidx.at[rk2]], rbuf, gsem)

        section(0, rows0, gs0)
        section(1, rows1, gs1)
        section(2, rows0, gs0)
        section(3, rows1, gs1)
        return 0

    lax.fori_loop(0, KCH // 4, body, 0)
    plsc.subcore_barrier()

    pltpu.sync_copy(acc.at[pl.ds(s * RPT, RPT)],
                    out_hbm.at[c, pl.ds(s * RPT, RPT)])


@jax.jit
def _sc_agg(feat, edge4, zeros):
    mesh = plsc.VectorSubcoreMesh(core_axis_name="c", subcore_axis_name="s",
                                  num_cores=NC, num_subcores=NS)
    f = pl.kernel(
        _sc_agg_body,
        out_type=jax.ShapeDtypeStruct((NC, NPAD, D), jnp.float32),
        mesh=mesh,
        scratch_types=[
            pltpu.VMEM((16, CHUNK), jnp.int32),
            pltpu.VMEM((KCH, CHUNK), jnp.int32),
            pltpu.VMEM((CHUNK, D), jnp.float32),
            pltpu.VMEM((CHUNK, D), jnp.float32),
            pltpu.SemaphoreType.DMA,
            pltpu.SemaphoreType.DMA,
            pltpu.SemaphoreType.DMA,
            pltpu.VMEM_SHARED((NPAD, D), jnp.float32),
        ],
    )
    return f(feat, edge4, zeros)


def _tc_self_body(feat_ref, ws_ref, out_ref):
    out_ref[...] = lax.dot(feat_ref[...], ws_ref[...],
                           preferred_element_type=jnp.float32)


@jax.jit
def _tc_self(feat, ws):
    blk = 2000
    return pl.pallas_call(
        _tc_self_body,
        grid=(N // blk,),
        in_specs=[
            pl.BlockSpec((blk, D), lambda i: (i, 0)),
            pl.BlockSpec((D, D), lambda i: (0, 0)),
        ],
        out_specs=pl.BlockSpec((blk, D), lambda i: (i, 0)),
        out_shape=jax.ShapeDtypeStruct((N, D), jnp.float32),
    )(feat, ws)


def _tc_combine_body(p_ref, base_ref, norm_ref, wn_ref, out_ref):
    ssum = p_ref[0] + p_ref[1]
    agg = lax.dot(ssum, wn_ref[...], preferred_element_type=jnp.float32)
    out_ref[...] = agg * norm_ref[...] + base_ref[...]


@jax.jit
def _tc_combine(partials, base, norm, wn):
    blk = 2000
    return pl.pallas_call(
        _tc_combine_body,
        grid=(N // blk,),
        in_specs=[
            pl.BlockSpec((NC, blk, D), lambda i: (0, i, 0)),
            pl.BlockSpec((blk, D), lambda i: (i, 0)),
            pl.BlockSpec((blk, 1), lambda i: (i, 0)),
            pl.BlockSpec((D, D), lambda i: (0, 0)),
        ],
        out_specs=pl.BlockSpec((blk, D), lambda i: (i, 0)),
        out_shape=jax.ShapeDtypeStruct((N, D), jnp.float32),
    )(partials, base, norm, wn)


def kernel(feat, edge_index, norm_data, weight_n, weight_s):
    edge4 = edge_index.astype(jnp.int32).reshape(2, NW, KCH, CHUNK)
    zeros = jnp.zeros((RPT, D), jnp.float32)
    base = _tc_self(feat, weight_s)
    partials = _sc_agg(feat, edge4, zeros)
    return _tc_combine(partials, base, norm_data, weight_n)
